# baseline (speedup 1.0000x reference)
import jax
import jax.numpy as jnp
from jax.experimental import pallas as pl


def _gin_kernel(x_ref, adj_ref, w1_ref, b1_ref, w2_ref, b2_ref,
                eps1_ref, eps2_ref, out_ref):
    bb, n, _ = x_ref.shape
    h_dim = w1_ref.shape[1]
    eps1 = eps1_ref[0, 0]
    eps2 = eps2_ref[0, 0]

    xflat = x_ref[...].reshape(bb * n, x_ref.shape[2])
    yflat = jax.lax.dot_general(xflat, w1_ref[...], (((1,), (0,)), ((), ())),
                                preferred_element_type=jnp.float32)

    for i in range(bb):
        adjf = (adj_ref[i] != 0).astype(jnp.float32)
        y = yflat[i * n:(i + 1) * n]
        z = jax.lax.dot_general(adjf, y, (((0,), (0,)), ((), ())),
                                preferred_element_type=jnp.float32)
        h = jnp.maximum((1.0 + eps1) * y + z + b1_ref[...], 0.0)

        w = (1.0 + eps2) + jnp.sum(adjf, axis=1, keepdims=True)
        m = jnp.sum(w * h, axis=0, keepdims=True) * (1.0 / n)

        o = jax.lax.dot_general(m, w2_ref[...], (((1,), (0,)), ((), ())),
                                preferred_element_type=jnp.float32)
        out_ref[i] = o + b2_ref[...]


def kernel(x, adj, W1, b1, W2, b2, eps1, eps2):
    B, N, C = x.shape
    H = W1.shape[1]
    Co = W2.shape[1]
    b1r = b1.reshape(1, H)
    b2r = b2.reshape(1, Co)
    e1 = jnp.asarray(eps1, jnp.float32).reshape(1, 1)
    e2 = jnp.asarray(eps2, jnp.float32).reshape(1, 1)

    BB = 16
    out = pl.pallas_call(
        _gin_kernel,
        grid=(B // BB,),
        in_specs=[
            pl.BlockSpec((BB, N, C), lambda b: (b, 0, 0)),
            pl.BlockSpec((BB, N, N), lambda b: (b, 0, 0)),
            pl.BlockSpec((C, H), lambda b: (0, 0)),
            pl.BlockSpec((1, H), lambda b: (0, 0)),
            pl.BlockSpec((H, Co), lambda b: (0, 0)),
            pl.BlockSpec((1, Co), lambda b: (0, 0)),
            pl.BlockSpec((1, 1), lambda b: (0, 0)),
            pl.BlockSpec((1, 1), lambda b: (0, 0)),
        ],
        out_specs=pl.BlockSpec((BB, 1, Co), lambda b: (b, 0, 0)),
        out_shape=jax.ShapeDtypeStruct((B, 1, Co), jnp.float32),
    )(x, adj, W1, b1r, W2, b2r, e1, e2)
    return out.reshape(B, Co)


# speedup vs baseline: 1.1714x; 1.1714x over previous
import jax
import jax.numpy as jnp
from jax.experimental import pallas as pl


def _gin_kernel(x_ref, adj_ref, w1_ref, b1_ref, w2_ref, b2_ref,
                eps1_ref, eps2_ref, out_ref):
    bb, n, _ = x_ref.shape
    h_dim = w1_ref.shape[1]
    eps1 = eps1_ref[0, 0]
    eps2 = eps2_ref[0, 0]

    xflat = x_ref[...].reshape(bb * n, x_ref.shape[2])
    yflat = jax.lax.dot_general(xflat, w1_ref[...], (((1,), (0,)), ((), ())),
                                preferred_element_type=jnp.float32)

    for i in range(bb):
        adjf = (adj_ref[i] != 0).astype(jnp.float32)
        y = yflat[i * n:(i + 1) * n]
        z = jax.lax.dot_general(adjf, y, (((0,), (0,)), ((), ())),
                                preferred_element_type=jnp.float32)
        h = jnp.maximum((1.0 + eps1) * y + z + b1_ref[...], 0.0)

        w = (1.0 + eps2) + jnp.sum(adjf, axis=1, keepdims=True)
        m = jnp.sum(w * h, axis=0, keepdims=True) * (1.0 / n)

        o = jax.lax.dot_general(m, w2_ref[...], (((1,), (0,)), ((), ())),
                                preferred_element_type=jnp.float32)
        out_ref[i] = o + b2_ref[...]


def kernel(x, adj, W1, b1, W2, b2, eps1, eps2):
    B, N, C = x.shape
    H = W1.shape[1]
    Co = W2.shape[1]
    b1r = b1.reshape(1, H)
    b2r = b2.reshape(1, Co)
    e1 = jnp.asarray(eps1, jnp.float32).reshape(1, 1)
    e2 = jnp.asarray(eps2, jnp.float32).reshape(1, 1)

    BB = 8
    out = pl.pallas_call(
        _gin_kernel,
        grid=(B // BB,),
        in_specs=[
            pl.BlockSpec((BB, N, C), lambda b: (b, 0, 0)),
            pl.BlockSpec((BB, N, N), lambda b: (b, 0, 0)),
            pl.BlockSpec((C, H), lambda b: (0, 0)),
            pl.BlockSpec((1, H), lambda b: (0, 0)),
            pl.BlockSpec((H, Co), lambda b: (0, 0)),
            pl.BlockSpec((1, Co), lambda b: (0, 0)),
            pl.BlockSpec((1, 1), lambda b: (0, 0)),
            pl.BlockSpec((1, 1), lambda b: (0, 0)),
        ],
        out_specs=pl.BlockSpec((BB, 1, Co), lambda b: (b, 0, 0)),
        out_shape=jax.ShapeDtypeStruct((B, 1, Co), jnp.float32),
    )(x, adj, W1, b1r, W2, b2r, e1, e2)
    return out.reshape(B, Co)


# speedup vs baseline: 1.3622x; 1.1628x over previous
import jax
import jax.numpy as jnp
from jax.experimental import pallas as pl


def _gin_kernel(x_ref, adj_ref, w1_ref, b1_ref, w2_ref, b2_ref,
                eps1_ref, eps2_ref, out_ref):
    bb, n, _ = x_ref.shape
    h_dim = w1_ref.shape[1]
    eps1 = eps1_ref[0, 0]
    eps2 = eps2_ref[0, 0]

    xflat = x_ref[...].reshape(bb * n, x_ref.shape[2])
    yflat = jax.lax.dot_general(xflat, w1_ref[...], (((1,), (0,)), ((), ())),
                                preferred_element_type=jnp.float32)

    adjall = (adj_ref[...] != 0).astype(jnp.float32)

    zs = [
        jax.lax.dot_general(adjall[i], yflat[i * n:(i + 1) * n],
                            (((0,), (0,)), ((), ())),
                            preferred_element_type=jnp.float32)
        for i in range(bb)
    ]
    zflat = jnp.concatenate(zs, axis=0)
    hflat = jnp.maximum((1.0 + eps1) * yflat + zflat + b1_ref[...], 0.0)

    w3 = (1.0 + eps2) + jnp.sum(adjall, axis=2, keepdims=True)
    h3 = hflat.reshape(bb, n, h_dim)
    mm = jnp.sum(h3 * w3, axis=1) * (1.0 / n)

    oo = jax.lax.dot_general(mm, w2_ref[...], (((1,), (0,)), ((), ())),
                             preferred_element_type=jnp.float32)
    out_ref[...] = (oo + b2_ref[...]).reshape(bb, 1, oo.shape[1])


def kernel(x, adj, W1, b1, W2, b2, eps1, eps2):
    B, N, C = x.shape
    H = W1.shape[1]
    Co = W2.shape[1]
    b1r = b1.reshape(1, H)
    b2r = b2.reshape(1, Co)
    e1 = jnp.asarray(eps1, jnp.float32).reshape(1, 1)
    e2 = jnp.asarray(eps2, jnp.float32).reshape(1, 1)

    BB = 8
    out = pl.pallas_call(
        _gin_kernel,
        grid=(B // BB,),
        in_specs=[
            pl.BlockSpec((BB, N, C), lambda b: (b, 0, 0)),
            pl.BlockSpec((BB, N, N), lambda b: (b, 0, 0)),
            pl.BlockSpec((C, H), lambda b: (0, 0)),
            pl.BlockSpec((1, H), lambda b: (0, 0)),
            pl.BlockSpec((H, Co), lambda b: (0, 0)),
            pl.BlockSpec((1, Co), lambda b: (0, 0)),
            pl.BlockSpec((1, 1), lambda b: (0, 0)),
            pl.BlockSpec((1, 1), lambda b: (0, 0)),
        ],
        out_specs=pl.BlockSpec((BB, 1, Co), lambda b: (b, 0, 0)),
        out_shape=jax.ShapeDtypeStruct((B, 1, Co), jnp.float32),
    )(x, adj, W1, b1r, W2, b2r, e1, e2)
    return out.reshape(B, Co)
